# baseline (speedup 1.0000x reference)
import functools

import jax
import jax.numpy as jnp
from jax import lax
from jax.experimental import pallas as pl
from jax.experimental.pallas import tpu as pltpu
from jax.experimental.pallas import tpu_sc as plsc

NUM_TABLES = 26
VOCAB = 100000
DIM = 32
BATCH = 4096
L = 20

NUM_GROUPS = 2
GT = NUM_TABLES // NUM_GROUPS

S = 25088
LSTEP = 3584
NSTEP = S // LSTEP + 1
LINES_T = NSTEP * LSTEP
TAIL0 = VOCAB - 32
OV3 = TAIL0 - LSTEP
B3 = 3 * S + (NSTEP - 2) * LSTEP

NUM_WORKERS = 32
BLOCK_BAGS = 64
BAGS_PER_WORKER = BATCH // NUM_WORKERS
BLOCKS_PER_WORKER = BAGS_PER_WORKER // BLOCK_BAGS
HALF = 16


def _tc_pack(t2g):
    def body(in_ref, out_ref):
        i32 = jnp.int32
        c = pl.program_id(1)
        base = c * i32(LSTEP)

        @pl.when(c < NSTEP - 1)
        def _main():
            pieces = []
            for k in range(4):
                off = k * i32(S) + base
                if k == 3:
                    off = jnp.where(c == i32(NSTEP - 2), i32(OV3), off)
                off = pl.multiple_of(off, 128)
                pieces.append(in_ref[0, :, pl.ds(off, LSTEP)])
            out_ref[0] = jnp.concatenate(pieces, axis=0).T

        @pl.when(c == NSTEP - 1)
        def _tail():
            xs = in_ref[0, :, pl.ds(TAIL0, 32)]
            val = jnp.concatenate(
                [xs.T, jnp.zeros((32, 96), jnp.float32)], axis=1
            )
            out_ref[0] = jnp.concatenate(
                [val, jnp.zeros((LSTEP - 32, 128), jnp.float32)], axis=0
            )

    return pl.pallas_call(
        body,
        grid=(GT, NSTEP),
        in_specs=[
            pl.BlockSpec((1, DIM, VOCAB), lambda t, c: (t, t * 0, t * 0)),
        ],
        out_specs=pl.BlockSpec(
            (1, LSTEP, 128), lambda t, c: (t, c, t * 0)
        ),
        out_shape=jax.ShapeDtypeStruct((GT, LINES_T, 128), jnp.float32),
        compiler_params=pltpu.CompilerParams(
            vmem_limit_bytes=48 * 1024 * 1024
        ),
    )(t2g)


def _sc_embedding_bag(table_flat, idx3):
    mesh = plsc.VectorSubcoreMesh(core_axis_name="c", subcore_axis_name="s")

    @functools.partial(
        pl.kernel,
        out_type=jax.ShapeDtypeStruct((BATCH, GT * DIM), jnp.float32),
        mesh=mesh,
        scratch_types=[
            pltpu.VMEM((L, BLOCK_BAGS), jnp.int32),
            pltpu.VMEM((L * BLOCK_BAGS, DIM), jnp.float32),
            pltpu.VMEM((BLOCK_BAGS, GT * DIM), jnp.float32),
            pltpu.SemaphoreType.DMA,
        ],
        compiler_params=pltpu.CompilerParams(use_tc_tiling_on_sc=False),
    )
    def k(table_hbm, idx_hbm, out_hbm, idx_v, rows_v, out_v, sem):
        i32 = jnp.int32
        wid = lax.axis_index("s") * i32(2) + lax.axis_index("c")

        for kb in range(BLOCKS_PER_WORKER):
            b0 = wid * i32(BAGS_PER_WORKER) + i32(kb * BLOCK_BAGS)

            def _table(t, _):
                pltpu.sync_copy(
                    idx_hbm.at[t, :, pl.ds(b0, BLOCK_BAGS)], idx_v
                )
                for j in range(L):
                    pltpu.async_copy(
                        table_hbm.at[idx_v.at[i32(j)]],
                        rows_v.at[pl.ds(j * BLOCK_BAGS, BLOCK_BAGS)],
                        sem,
                    )
                for j in range(L):
                    pltpu.make_async_copy(
                        table_hbm.at[idx_v.at[i32(j)]],
                        rows_v.at[pl.ds(j * BLOCK_BAGS, BLOCK_BAGS)],
                        sem,
                    ).wait()

                col = t * i32(DIM)

                def _bag(i, _):
                    lo = rows_v[i, pl.ds(0, HALF)]
                    hi = rows_v[i, pl.ds(HALF, HALF)]
                    for l in range(1, L):
                        r = i + i32(l * BLOCK_BAGS)
                        lo = lo + rows_v[r, pl.ds(0, HALF)]
                        hi = hi + rows_v[r, pl.ds(HALF, HALF)]
                    out_v[i, pl.ds(col, HALF)] = lo
                    out_v[i, pl.ds(col + i32(HALF), HALF)] = hi
                    return _

                lax.fori_loop(i32(0), i32(BLOCK_BAGS), _bag, None)
                return _

            lax.fori_loop(i32(0), i32(GT), _table, None)

            pltpu.sync_copy(out_v, out_hbm.at[pl.ds(b0, BLOCK_BAGS), :])

    return k(table_flat, idx3)


def kernel(indices, tables):
    v = jnp.transpose(indices, (0, 2, 1)).astype(jnp.int32)
    g = jnp.where(
        v < 3 * S,
        (v % S) * 4 + v // S,
        jnp.where(
            v < B3,
            (v - 3 * S) * 4 + 3,
            (v - OV3 + (NSTEP - 2) * LSTEP) * 4
            + jnp.where(v < TAIL0, 3, 0),
        ),
    )
    t_local = (jnp.arange(NUM_TABLES, dtype=jnp.int32) % GT)[:, None, None]
    g = g + t_local * (LINES_T * 4)

    t2 = jnp.transpose(tables, (0, 2, 1))
    outs = []
    for grp in range(NUM_GROUPS):
        packed = _tc_pack(t2[grp * GT:(grp + 1) * GT])
        table_flat = packed.reshape(GT * LINES_T * 4, DIM)
        outs.append(
            _sc_embedding_bag(table_flat, g[grp * GT:(grp + 1) * GT])
        )
    return jnp.concatenate(outs, axis=1)


# speedup vs baseline: 1.3358x; 1.3358x over previous
import functools

import jax
import jax.numpy as jnp
from jax import lax
from jax.experimental import pallas as pl
from jax.experimental.pallas import tpu as pltpu
from jax.experimental.pallas import tpu_sc as plsc

NUM_TABLES = 26
VOCAB = 100000
DIM = 32
BATCH = 4096
L = 20

NUM_GROUPS = 2
GT = NUM_TABLES // NUM_GROUPS

S = 25088
LSTEP = 3584
NSTEP = S // LSTEP + 1
LINES_T = NSTEP * LSTEP
TAIL0 = VOCAB - 32
OV3 = TAIL0 - LSTEP
B3 = 3 * S + (NSTEP - 2) * LSTEP

NUM_WORKERS = 32
BLOCK_BAGS = 64
BAGS_PER_WORKER = BATCH // NUM_WORKERS
BLOCKS_PER_WORKER = BAGS_PER_WORKER // BLOCK_BAGS
HALF = 16


def _tc_pack(t2, grp):
    def body(in_ref, out_ref):
        i32 = jnp.int32
        c = pl.program_id(1)
        base = c * i32(LSTEP)

        @pl.when(c < NSTEP - 1)
        def _main():
            pieces = []
            for k in range(4):
                off = k * i32(S) + base
                if k == 3:
                    off = jnp.where(c == i32(NSTEP - 2), i32(OV3), off)
                off = pl.multiple_of(off, 128)
                pieces.append(in_ref[0, :, pl.ds(off, LSTEP)])
            out_ref[0] = jnp.concatenate(pieces, axis=0).T

        @pl.when(c == NSTEP - 1)
        def _tail():
            xs = in_ref[0, :, pl.ds(TAIL0, 32)]
            val = jnp.concatenate(
                [xs.T, jnp.zeros((32, 96), jnp.float32)], axis=1
            )
            out_ref[0] = jnp.concatenate(
                [val, jnp.zeros((LSTEP - 32, 128), jnp.float32)], axis=0
            )

    return pl.pallas_call(
        body,
        grid=(GT, NSTEP),
        in_specs=[
            pl.BlockSpec(
                (1, DIM, VOCAB), lambda t, c: (t + grp * GT, t * 0, t * 0)
            ),
        ],
        out_specs=pl.BlockSpec(
            (1, LSTEP, 128), lambda t, c: (t, c, t * 0)
        ),
        out_shape=jax.ShapeDtypeStruct((GT, LINES_T, 128), jnp.float32),
        compiler_params=pltpu.CompilerParams(
            vmem_limit_bytes=48 * 1024 * 1024
        ),
    )(t2)


def _sc_embedding_bag(table_flat, idx3, grp):
    mesh = plsc.VectorSubcoreMesh(core_axis_name="c", subcore_axis_name="s")

    @functools.partial(
        pl.kernel,
        out_type=jax.ShapeDtypeStruct((BATCH, GT * DIM), jnp.float32),
        mesh=mesh,
        scratch_types=[
            pltpu.VMEM((L, BLOCK_BAGS), jnp.int32),
            pltpu.VMEM((L * BLOCK_BAGS, DIM), jnp.float32),
            pltpu.VMEM((BLOCK_BAGS, GT * DIM), jnp.float32),
            pltpu.SemaphoreType.DMA,
        ],
        compiler_params=pltpu.CompilerParams(use_tc_tiling_on_sc=False),
    )
    def k(table_hbm, idx_hbm, out_hbm, idx_v, rows_v, out_v, sem):
        i32 = jnp.int32
        wid = lax.axis_index("s") * i32(2) + lax.axis_index("c")

        for kb in range(BLOCKS_PER_WORKER):
            b0 = wid * i32(BAGS_PER_WORKER) + i32(kb * BLOCK_BAGS)

            def _table(t, _):
                pltpu.sync_copy(
                    idx_hbm.at[t + i32(grp * GT), :, pl.ds(b0, BLOCK_BAGS)],
                    idx_v,
                )
                for j in range(L):
                    pltpu.async_copy(
                        table_hbm.at[idx_v.at[i32(j)]],
                        rows_v.at[pl.ds(j * BLOCK_BAGS, BLOCK_BAGS)],
                        sem,
                    )
                for j in range(L):
                    pltpu.make_async_copy(
                        table_hbm.at[idx_v.at[i32(j)]],
                        rows_v.at[pl.ds(j * BLOCK_BAGS, BLOCK_BAGS)],
                        sem,
                    ).wait()

                col = t * i32(DIM)

                def _bag(i, _):
                    lo = rows_v[i, pl.ds(0, HALF)]
                    hi = rows_v[i, pl.ds(HALF, HALF)]
                    for l in range(1, L):
                        r = i + i32(l * BLOCK_BAGS)
                        lo = lo + rows_v[r, pl.ds(0, HALF)]
                        hi = hi + rows_v[r, pl.ds(HALF, HALF)]
                    out_v[i, pl.ds(col, HALF)] = lo
                    out_v[i, pl.ds(col + i32(HALF), HALF)] = hi
                    return _

                lax.fori_loop(i32(0), i32(BLOCK_BAGS), _bag, None)
                return _

            lax.fori_loop(i32(0), i32(GT), _table, None)

            pltpu.sync_copy(out_v, out_hbm.at[pl.ds(b0, BLOCK_BAGS), :])

    return k(table_flat, idx3)


def kernel(indices, tables):
    v = jnp.transpose(indices, (0, 2, 1)).astype(jnp.int32)
    g = jnp.where(
        v < 3 * S,
        (v % S) * 4 + v // S,
        jnp.where(
            v < B3,
            (v - 3 * S) * 4 + 3,
            (v - OV3 + (NSTEP - 2) * LSTEP) * 4
            + jnp.where(v < TAIL0, 3, 0),
        ),
    )
    t_local = (jnp.arange(NUM_TABLES, dtype=jnp.int32) % GT)[:, None, None]
    g = g + t_local * (LINES_T * 4)

    t2 = jnp.transpose(tables, (0, 2, 1))
    outs = []
    for grp in range(NUM_GROUPS):
        packed = _tc_pack(t2, grp)
        table_flat = packed.reshape(GT * LINES_T * 4, DIM)
        outs.append(_sc_embedding_bag(table_flat, g, grp))
    return jnp.concatenate(outs, axis=1)


# speedup vs baseline: 1.4303x; 1.0707x over previous
import functools

import jax
import jax.numpy as jnp
from jax import lax
from jax.experimental import pallas as pl
from jax.experimental.pallas import tpu as pltpu
from jax.experimental.pallas import tpu_sc as plsc

NUM_TABLES = 26
VOCAB = 100000
DIM = 32
BATCH = 4096
L = 20

GROUP_SIZES = (9, 9, 8)
GROUP_STARTS = (0, 9, 18)

S = 25088
LSTEP = 3584
NSTEP = S // LSTEP + 1
LINES_T = NSTEP * LSTEP
TAIL0 = VOCAB - 32
OV3 = TAIL0 - LSTEP
B3 = 3 * S + (NSTEP - 2) * LSTEP

NUM_WORKERS = 32
BLOCK_BAGS = 64
BAGS_PER_WORKER = BATCH // NUM_WORKERS
BLOCKS_PER_WORKER = BAGS_PER_WORKER // BLOCK_BAGS
HALF = 16


def _tc_pack(t2, gt, start):
    def body(in_ref, out_ref):
        i32 = jnp.int32
        c = pl.program_id(1)
        base = c * i32(LSTEP)

        @pl.when(c < NSTEP - 1)
        def _main():
            pieces = []
            for k in range(4):
                off = k * i32(S) + base
                if k == 3:
                    off = jnp.where(c == i32(NSTEP - 2), i32(OV3), off)
                off = pl.multiple_of(off, 128)
                pieces.append(in_ref[0, :, pl.ds(off, LSTEP)])
            out_ref[0] = jnp.concatenate(pieces, axis=0).T

        @pl.when(c == NSTEP - 1)
        def _tail():
            xs = in_ref[0, :, pl.ds(TAIL0, 32)]
            val = jnp.concatenate(
                [xs.T, jnp.zeros((32, 96), jnp.float32)], axis=1
            )
            out_ref[0] = jnp.concatenate(
                [val, jnp.zeros((LSTEP - 32, 128), jnp.float32)], axis=0
            )

    return pl.pallas_call(
        body,
        grid=(gt, NSTEP),
        in_specs=[
            pl.BlockSpec(
                (1, DIM, VOCAB), lambda t, c: (t + start, t * 0, t * 0)
            ),
        ],
        out_specs=pl.BlockSpec(
            (1, LSTEP, 128), lambda t, c: (t, c, t * 0)
        ),
        out_shape=jax.ShapeDtypeStruct((gt, LINES_T, 128), jnp.float32),
        compiler_params=pltpu.CompilerParams(
            vmem_limit_bytes=48 * 1024 * 1024
        ),
    )(t2)


def _sc_embedding_bag(table_flat, idx3, gt, start):
    mesh = plsc.VectorSubcoreMesh(core_axis_name="c", subcore_axis_name="s")

    @functools.partial(
        pl.kernel,
        out_type=jax.ShapeDtypeStruct((BATCH, gt * DIM), jnp.float32),
        mesh=mesh,
        scratch_types=[
            pltpu.VMEM((L, BLOCK_BAGS), jnp.int32),
            pltpu.VMEM((L * BLOCK_BAGS, DIM), jnp.float32),
            pltpu.VMEM((BLOCK_BAGS, gt * DIM), jnp.float32),
            pltpu.SemaphoreType.DMA,
        ],
        compiler_params=pltpu.CompilerParams(use_tc_tiling_on_sc=False),
    )
    def k(table_hbm, idx_hbm, out_hbm, idx_v, rows_v, out_v, sem):
        i32 = jnp.int32
        wid = lax.axis_index("s") * i32(2) + lax.axis_index("c")

        for kb in range(BLOCKS_PER_WORKER):
            b0 = wid * i32(BAGS_PER_WORKER) + i32(kb * BLOCK_BAGS)

            def _table(t, _):
                pltpu.sync_copy(
                    idx_hbm.at[t + i32(start), :, pl.ds(b0, BLOCK_BAGS)],
                    idx_v,
                )
                for j in range(L):
                    pltpu.async_copy(
                        table_hbm.at[idx_v.at[i32(j)]],
                        rows_v.at[pl.ds(j * BLOCK_BAGS, BLOCK_BAGS)],
                        sem,
                    )
                for j in range(L):
                    pltpu.make_async_copy(
                        table_hbm.at[idx_v.at[i32(j)]],
                        rows_v.at[pl.ds(j * BLOCK_BAGS, BLOCK_BAGS)],
                        sem,
                    ).wait()

                col = t * i32(DIM)

                def _bag(i, _):
                    lo = rows_v[i, pl.ds(0, HALF)]
                    hi = rows_v[i, pl.ds(HALF, HALF)]
                    for l in range(1, L):
                        r = i + i32(l * BLOCK_BAGS)
                        lo = lo + rows_v[r, pl.ds(0, HALF)]
                        hi = hi + rows_v[r, pl.ds(HALF, HALF)]
                    out_v[i, pl.ds(col, HALF)] = lo
                    out_v[i, pl.ds(col + i32(HALF), HALF)] = hi
                    return _

                lax.fori_loop(i32(0), i32(BLOCK_BAGS), _bag, None)
                return _

            lax.fori_loop(i32(0), i32(gt), _table, None)

            pltpu.sync_copy(out_v, out_hbm.at[pl.ds(b0, BLOCK_BAGS), :])

    return k(table_flat, idx3)


def kernel(indices, tables):
    v = jnp.transpose(indices, (0, 2, 1)).astype(jnp.int32)
    g = jnp.where(
        v < 3 * S,
        (v % S) * 4 + v // S,
        jnp.where(
            v < B3,
            (v - 3 * S) * 4 + 3,
            (v - OV3 + (NSTEP - 2) * LSTEP) * 4
            + jnp.where(v < TAIL0, 3, 0),
        ),
    )
    t_glob = jnp.arange(NUM_TABLES, dtype=jnp.int32)
    t_start = jnp.zeros((NUM_TABLES,), jnp.int32)
    for st, sz in zip(GROUP_STARTS, GROUP_SIZES):
        t_start = jnp.where(t_glob >= st, st, t_start)
    t_local = (t_glob - t_start)[:, None, None]
    g = g + t_local * (LINES_T * 4)

    t2 = jnp.transpose(tables, (0, 2, 1))
    outs = []
    for st, sz in zip(GROUP_STARTS, GROUP_SIZES):
        packed = _tc_pack(t2, sz, st)
        table_flat = packed.reshape(sz * LINES_T * 4, DIM)
        outs.append(_sc_embedding_bag(table_flat, g, sz, st))
    return jnp.concatenate(outs, axis=1)


# speedup vs baseline: 1.4796x; 1.0345x over previous
import functools

import jax
import jax.numpy as jnp
from jax import lax
from jax.experimental import pallas as pl
from jax.experimental.pallas import tpu as pltpu
from jax.experimental.pallas import tpu_sc as plsc

NUM_TABLES = 26
VOCAB = 100000
DIM = 32
BATCH = 4096
L = 20

GROUP_SIZES = (9, 9, 8)
GROUP_STARTS = (0, 9, 18)

S = 25088
LSTEP = 3584
NSTEP = S // LSTEP + 1
LINES_T = NSTEP * LSTEP
TAIL0 = VOCAB - 32
OV3 = TAIL0 - LSTEP
B3 = 3 * S + (NSTEP - 2) * LSTEP

NUM_WORKERS = 32
BLOCK_BAGS = 64
BAGS_PER_WORKER = BATCH // NUM_WORKERS
BLOCKS_PER_WORKER = BAGS_PER_WORKER // BLOCK_BAGS
HALF = 16


def _tc_pack(t2, gt, start):
    def body(in_ref, out_ref):
        i32 = jnp.int32
        c = pl.program_id(1)
        base = c * i32(LSTEP)

        @pl.when(c < NSTEP - 1)
        def _main():
            pieces = []
            for k in range(4):
                off = k * i32(S) + base
                if k == 3:
                    off = jnp.where(c == i32(NSTEP - 2), i32(OV3), off)
                off = pl.multiple_of(off, 128)
                pieces.append(in_ref[0, :, pl.ds(off, LSTEP)])
            out_ref[0] = jnp.concatenate(pieces, axis=0).T

        @pl.when(c == NSTEP - 1)
        def _tail():
            xs = in_ref[0, :, pl.ds(TAIL0, 32)]
            val = jnp.concatenate(
                [xs.T, jnp.zeros((32, 96), jnp.float32)], axis=1
            )
            out_ref[0] = jnp.concatenate(
                [val, jnp.zeros((LSTEP - 32, 128), jnp.float32)], axis=0
            )

    return pl.pallas_call(
        body,
        grid=(gt, NSTEP),
        in_specs=[
            pl.BlockSpec(
                (1, DIM, VOCAB), lambda t, c: (t + start, t * 0, t * 0)
            ),
        ],
        out_specs=pl.BlockSpec(
            (1, LSTEP, 128), lambda t, c: (t, c, t * 0)
        ),
        out_shape=jax.ShapeDtypeStruct((gt, LINES_T, 128), jnp.float32),
        compiler_params=pltpu.CompilerParams(
            vmem_limit_bytes=48 * 1024 * 1024
        ),
    )(t2)


def _sc_embedding_bag(table_flat, idx3, gt, start):
    mesh = plsc.VectorSubcoreMesh(core_axis_name="c", subcore_axis_name="s")

    @functools.partial(
        pl.kernel,
        out_type=jax.ShapeDtypeStruct((BATCH, gt * DIM), jnp.float32),
        mesh=mesh,
        scratch_types=[
            pltpu.VMEM((L, BLOCK_BAGS), jnp.int32),
            pltpu.VMEM((L, BLOCK_BAGS), jnp.int32),
            pltpu.VMEM((L * BLOCK_BAGS, DIM), jnp.float32),
            pltpu.VMEM((L * BLOCK_BAGS, DIM), jnp.float32),
            pltpu.VMEM((BLOCK_BAGS, gt * DIM), jnp.float32),
            pltpu.SemaphoreType.DMA,
            pltpu.SemaphoreType.DMA,
            pltpu.SemaphoreType.DMA,
            pltpu.SemaphoreType.DMA,
        ],
        compiler_params=pltpu.CompilerParams(use_tc_tiling_on_sc=False),
    )
    def k(table_hbm, idx_hbm, out_hbm,
          idx0, idx1, rows0, rows1, out_v, si0, si1, sg0, sg1):
        i32 = jnp.int32
        wid = lax.axis_index("s") * i32(2) + lax.axis_index("c")
        idx_b = (idx0, idx1)
        rows_b = (rows0, rows1)
        si_b = (si0, si1)
        sg_b = (sg0, sg1)

        def idx_copy(t, p):
            return pltpu.make_async_copy(
                idx_hbm.at[i32(t + start), :, pl.ds(b0, BLOCK_BAGS)],
                idx_b[p],
                si_b[p],
            )

        def gathers(t, p):
            for j in range(L):
                pltpu.async_copy(
                    table_hbm.at[idx_b[p].at[i32(j)]],
                    rows_b[p].at[pl.ds(j * BLOCK_BAGS, BLOCK_BAGS)],
                    sg_b[p],
                )

        def wait_gathers(t, p):
            for j in range(L):
                pltpu.make_async_copy(
                    table_hbm.at[idx_b[p].at[i32(j)]],
                    rows_b[p].at[pl.ds(j * BLOCK_BAGS, BLOCK_BAGS)],
                    sg_b[p],
                ).wait()

        def pool(t, p):
            rows_v = rows_b[p]
            col = i32(t * DIM)

            def _bag(i, _):
                lo = rows_v[i, pl.ds(0, HALF)]
                hi = rows_v[i, pl.ds(HALF, HALF)]
                for l in range(1, L):
                    r = i + i32(l * BLOCK_BAGS)
                    lo = lo + rows_v[r, pl.ds(0, HALF)]
                    hi = hi + rows_v[r, pl.ds(HALF, HALF)]
                out_v[i, pl.ds(col, HALF)] = lo
                out_v[i, pl.ds(col + i32(HALF), HALF)] = hi
                return _

            lax.fori_loop(i32(0), i32(BLOCK_BAGS), _bag, None)

        for kb in range(BLOCKS_PER_WORKER):
            b0 = wid * i32(BAGS_PER_WORKER) + i32(kb * BLOCK_BAGS)

            idx_copy(0, 0).start()
            idx_copy(0, 0).wait()
            gathers(0, 0)
            if gt > 1:
                idx_copy(1, 1).start()
            for t in range(gt):
                p = t & 1
                wait_gathers(t, p)
                if t + 1 < gt:
                    idx_copy(t + 1, p ^ 1).wait()
                    gathers(t + 1, p ^ 1)
                if t + 2 < gt:
                    idx_copy(t + 2, p).start()
                pool(t, p)

            pltpu.sync_copy(out_v, out_hbm.at[pl.ds(b0, BLOCK_BAGS), :])

    return k(table_flat, idx3)


def kernel(indices, tables):
    v = jnp.transpose(indices, (0, 2, 1)).astype(jnp.int32)
    g = jnp.where(
        v < 3 * S,
        (v % S) * 4 + v // S,
        jnp.where(
            v < B3,
            (v - 3 * S) * 4 + 3,
            (v - OV3 + (NSTEP - 2) * LSTEP) * 4
            + jnp.where(v < TAIL0, 3, 0),
        ),
    )
    t_glob = jnp.arange(NUM_TABLES, dtype=jnp.int32)
    t_start = jnp.zeros((NUM_TABLES,), jnp.int32)
    for st, sz in zip(GROUP_STARTS, GROUP_SIZES):
        t_start = jnp.where(t_glob >= st, st, t_start)
    t_local = (t_glob - t_start)[:, None, None]
    g = g + t_local * (LINES_T * 4)

    t2 = jnp.transpose(tables, (0, 2, 1))
    outs = []
    for st, sz in zip(GROUP_STARTS, GROUP_SIZES):
        packed = _tc_pack(t2, sz, st)
        table_flat = packed.reshape(sz * LINES_T * 4, DIM)
        outs.append(_sc_embedding_bag(table_flat, g, sz, st))
    return jnp.concatenate(outs, axis=1)


# speedup vs baseline: 1.5040x; 1.0165x over previous
import functools

import jax
import jax.numpy as jnp
from jax import lax
from jax.experimental import pallas as pl
from jax.experimental.pallas import tpu as pltpu
from jax.experimental.pallas import tpu_sc as plsc

NUM_TABLES = 26
VOCAB = 100000
DIM = 32
BATCH = 4096
L = 20

GROUP_SIZES = (7, 7, 6, 6)
GROUP_STARTS = (0, 7, 14, 20)

S = 25088
LSTEP = 3584
NSTEP = S // LSTEP + 1
LINES_T = NSTEP * LSTEP
TAIL0 = VOCAB - 32
OV3 = TAIL0 - LSTEP
B3 = 3 * S + (NSTEP - 2) * LSTEP

NUM_WORKERS = 32
BLOCK_BAGS = 64
BAGS_PER_WORKER = BATCH // NUM_WORKERS
BLOCKS_PER_WORKER = BAGS_PER_WORKER // BLOCK_BAGS
HALF = 16


def _tc_pack(t2, gt, start):
    def body(in_ref, out_ref):
        i32 = jnp.int32
        c = pl.program_id(1)
        base = c * i32(LSTEP)

        @pl.when(c < NSTEP - 1)
        def _main():
            pieces = []
            for k in range(4):
                off = k * i32(S) + base
                if k == 3:
                    off = jnp.where(c == i32(NSTEP - 2), i32(OV3), off)
                off = pl.multiple_of(off, 128)
                pieces.append(in_ref[0, :, pl.ds(off, LSTEP)])
            out_ref[0] = jnp.concatenate(pieces, axis=0).T

        @pl.when(c == NSTEP - 1)
        def _tail():
            xs = in_ref[0, :, pl.ds(TAIL0, 32)]
            val = jnp.concatenate(
                [xs.T, jnp.zeros((32, 96), jnp.float32)], axis=1
            )
            out_ref[0] = jnp.concatenate(
                [val, jnp.zeros((LSTEP - 32, 128), jnp.float32)], axis=0
            )

    return pl.pallas_call(
        body,
        grid=(gt, NSTEP),
        in_specs=[
            pl.BlockSpec(
                (1, DIM, VOCAB), lambda t, c: (t + start, t * 0, t * 0)
            ),
        ],
        out_specs=pl.BlockSpec(
            (1, LSTEP, 128), lambda t, c: (t, c, t * 0)
        ),
        out_shape=jax.ShapeDtypeStruct((gt, LINES_T, 128), jnp.float32),
        compiler_params=pltpu.CompilerParams(
            vmem_limit_bytes=48 * 1024 * 1024
        ),
    )(t2)


def _sc_embedding_bag(table_flat, idx3, gt, start):
    mesh = plsc.VectorSubcoreMesh(core_axis_name="c", subcore_axis_name="s")

    @functools.partial(
        pl.kernel,
        out_type=jax.ShapeDtypeStruct((BATCH, gt * DIM), jnp.float32),
        mesh=mesh,
        scratch_types=[
            pltpu.VMEM((L, BLOCK_BAGS), jnp.int32),
            pltpu.VMEM((L, BLOCK_BAGS), jnp.int32),
            pltpu.VMEM((L * BLOCK_BAGS, DIM), jnp.float32),
            pltpu.VMEM((L * BLOCK_BAGS, DIM), jnp.float32),
            pltpu.VMEM((BLOCK_BAGS, gt * DIM), jnp.float32),
            pltpu.SemaphoreType.DMA,
            pltpu.SemaphoreType.DMA,
            pltpu.SemaphoreType.DMA,
            pltpu.SemaphoreType.DMA,
        ],
        compiler_params=pltpu.CompilerParams(use_tc_tiling_on_sc=False),
    )
    def k(table_hbm, idx_hbm, out_hbm,
          idx0, idx1, rows0, rows1, out_v, si0, si1, sg0, sg1):
        i32 = jnp.int32
        wid = lax.axis_index("s") * i32(2) + lax.axis_index("c")
        idx_b = (idx0, idx1)
        rows_b = (rows0, rows1)
        si_b = (si0, si1)
        sg_b = (sg0, sg1)

        def idx_copy(t, p):
            return pltpu.make_async_copy(
                idx_hbm.at[i32(t + start), :, pl.ds(b0, BLOCK_BAGS)],
                idx_b[p],
                si_b[p],
            )

        def gathers(t, p):
            for j in range(L):
                pltpu.async_copy(
                    table_hbm.at[idx_b[p].at[i32(j)]],
                    rows_b[p].at[pl.ds(j * BLOCK_BAGS, BLOCK_BAGS)],
                    sg_b[p],
                )

        def wait_gathers(t, p):
            for j in range(L):
                pltpu.make_async_copy(
                    table_hbm.at[idx_b[p].at[i32(j)]],
                    rows_b[p].at[pl.ds(j * BLOCK_BAGS, BLOCK_BAGS)],
                    sg_b[p],
                ).wait()

        def pool(t, p):
            rows_v = rows_b[p]
            col = i32(t * DIM)

            def _bag(i, _):
                lo = rows_v[i, pl.ds(0, HALF)]
                hi = rows_v[i, pl.ds(HALF, HALF)]
                for l in range(1, L):
                    r = i + i32(l * BLOCK_BAGS)
                    lo = lo + rows_v[r, pl.ds(0, HALF)]
                    hi = hi + rows_v[r, pl.ds(HALF, HALF)]
                out_v[i, pl.ds(col, HALF)] = lo
                out_v[i, pl.ds(col + i32(HALF), HALF)] = hi
                return _

            lax.fori_loop(i32(0), i32(BLOCK_BAGS), _bag, None)

        for kb in range(BLOCKS_PER_WORKER):
            b0 = wid * i32(BAGS_PER_WORKER) + i32(kb * BLOCK_BAGS)

            idx_copy(0, 0).start()
            idx_copy(0, 0).wait()
            gathers(0, 0)
            if gt > 1:
                idx_copy(1, 1).start()
            for t in range(gt):
                p = t & 1
                wait_gathers(t, p)
                if t + 1 < gt:
                    idx_copy(t + 1, p ^ 1).wait()
                    gathers(t + 1, p ^ 1)
                if t + 2 < gt:
                    idx_copy(t + 2, p).start()
                pool(t, p)

            pltpu.sync_copy(out_v, out_hbm.at[pl.ds(b0, BLOCK_BAGS), :])

    return k(table_flat, idx3)


def kernel(indices, tables):
    v = jnp.transpose(indices, (0, 2, 1)).astype(jnp.int32)
    g = jnp.where(
        v < 3 * S,
        (v % S) * 4 + v // S,
        jnp.where(
            v < B3,
            (v - 3 * S) * 4 + 3,
            (v - OV3 + (NSTEP - 2) * LSTEP) * 4
            + jnp.where(v < TAIL0, 3, 0),
        ),
    )
    t_glob = jnp.arange(NUM_TABLES, dtype=jnp.int32)
    t_start = jnp.zeros((NUM_TABLES,), jnp.int32)
    for st, sz in zip(GROUP_STARTS, GROUP_SIZES):
        t_start = jnp.where(t_glob >= st, st, t_start)
    t_local = (t_glob - t_start)[:, None, None]
    g = g + t_local * (LINES_T * 4)

    t2 = jnp.transpose(tables, (0, 2, 1))
    outs = []
    for st, sz in zip(GROUP_STARTS, GROUP_SIZES):
        packed = _tc_pack(t2, sz, st)
        table_flat = packed.reshape(sz * LINES_T * 4, DIM)
        outs.append(_sc_embedding_bag(table_flat, g, sz, st))
    return jnp.concatenate(outs, axis=1)


# speedup vs baseline: 1.5167x; 1.0085x over previous
import functools

import jax
import jax.numpy as jnp
from jax import lax
from jax.experimental import pallas as pl
from jax.experimental.pallas import tpu as pltpu
from jax.experimental.pallas import tpu_sc as plsc

NUM_TABLES = 26
VOCAB = 100000
DIM = 32
BATCH = 4096
L = 20

GROUP_SIZES = (8, 8, 7, 3)
GROUP_STARTS = (0, 8, 16, 23)

S = 25088
LSTEP = 3584
NSTEP = S // LSTEP + 1
LINES_T = NSTEP * LSTEP
TAIL0 = VOCAB - 32
OV3 = TAIL0 - LSTEP
B3 = 3 * S + (NSTEP - 2) * LSTEP

NUM_WORKERS = 32
BLOCK_BAGS = 64
BAGS_PER_WORKER = BATCH // NUM_WORKERS
BLOCKS_PER_WORKER = BAGS_PER_WORKER // BLOCK_BAGS
HALF = 16


def _tc_pack(t2, gt, start):
    def body(in_ref, out_ref):
        i32 = jnp.int32
        c = pl.program_id(1)
        base = c * i32(LSTEP)

        @pl.when(c < NSTEP - 1)
        def _main():
            pieces = []
            for k in range(4):
                off = k * i32(S) + base
                if k == 3:
                    off = jnp.where(c == i32(NSTEP - 2), i32(OV3), off)
                off = pl.multiple_of(off, 128)
                pieces.append(in_ref[0, :, pl.ds(off, LSTEP)])
            out_ref[0] = jnp.concatenate(pieces, axis=0).T

        @pl.when(c == NSTEP - 1)
        def _tail():
            xs = in_ref[0, :, pl.ds(TAIL0, 32)]
            val = jnp.concatenate(
                [xs.T, jnp.zeros((32, 96), jnp.float32)], axis=1
            )
            out_ref[0] = jnp.concatenate(
                [val, jnp.zeros((LSTEP - 32, 128), jnp.float32)], axis=0
            )

    return pl.pallas_call(
        body,
        grid=(gt, NSTEP),
        in_specs=[
            pl.BlockSpec(
                (1, DIM, VOCAB), lambda t, c: (t + start, t * 0, t * 0)
            ),
        ],
        out_specs=pl.BlockSpec(
            (1, LSTEP, 128), lambda t, c: (t, c, t * 0)
        ),
        out_shape=jax.ShapeDtypeStruct((gt, LINES_T, 128), jnp.float32),
        compiler_params=pltpu.CompilerParams(
            vmem_limit_bytes=48 * 1024 * 1024
        ),
    )(t2)


def _sc_embedding_bag(table_flat, idx3, gt, start):
    mesh = plsc.VectorSubcoreMesh(core_axis_name="c", subcore_axis_name="s")

    @functools.partial(
        pl.kernel,
        out_type=jax.ShapeDtypeStruct((BATCH, gt * DIM), jnp.float32),
        mesh=mesh,
        scratch_types=[
            pltpu.VMEM((L, BLOCK_BAGS), jnp.int32),
            pltpu.VMEM((L, BLOCK_BAGS), jnp.int32),
            pltpu.VMEM((L * BLOCK_BAGS, DIM), jnp.float32),
            pltpu.VMEM((L * BLOCK_BAGS, DIM), jnp.float32),
            pltpu.VMEM((BLOCK_BAGS, gt * DIM), jnp.float32),
            pltpu.SemaphoreType.DMA,
            pltpu.SemaphoreType.DMA,
            pltpu.SemaphoreType.DMA,
            pltpu.SemaphoreType.DMA,
        ],
        compiler_params=pltpu.CompilerParams(use_tc_tiling_on_sc=False),
    )
    def k(table_hbm, idx_hbm, out_hbm,
          idx0, idx1, rows0, rows1, out_v, si0, si1, sg0, sg1):
        i32 = jnp.int32
        wid = lax.axis_index("s") * i32(2) + lax.axis_index("c")
        idx_b = (idx0, idx1)
        rows_b = (rows0, rows1)
        si_b = (si0, si1)
        sg_b = (sg0, sg1)

        def idx_copy(t, p):
            return pltpu.make_async_copy(
                idx_hbm.at[i32(t + start), :, pl.ds(b0, BLOCK_BAGS)],
                idx_b[p],
                si_b[p],
            )

        def gathers(t, p):
            for j in range(L):
                pltpu.async_copy(
                    table_hbm.at[idx_b[p].at[i32(j)]],
                    rows_b[p].at[pl.ds(j * BLOCK_BAGS, BLOCK_BAGS)],
                    sg_b[p],
                )

        def wait_gathers(t, p):
            for j in range(L):
                pltpu.make_async_copy(
                    table_hbm.at[idx_b[p].at[i32(j)]],
                    rows_b[p].at[pl.ds(j * BLOCK_BAGS, BLOCK_BAGS)],
                    sg_b[p],
                ).wait()

        def pool(t, p):
            rows_v = rows_b[p]
            col = i32(t * DIM)

            def _bag(i, _):
                lo = rows_v[i, pl.ds(0, HALF)]
                hi = rows_v[i, pl.ds(HALF, HALF)]
                for l in range(1, L):
                    r = i + i32(l * BLOCK_BAGS)
                    lo = lo + rows_v[r, pl.ds(0, HALF)]
                    hi = hi + rows_v[r, pl.ds(HALF, HALF)]
                out_v[i, pl.ds(col, HALF)] = lo
                out_v[i, pl.ds(col + i32(HALF), HALF)] = hi
                return _

            lax.fori_loop(i32(0), i32(BLOCK_BAGS), _bag, None)

        for kb in range(BLOCKS_PER_WORKER):
            b0 = wid * i32(BAGS_PER_WORKER) + i32(kb * BLOCK_BAGS)

            idx_copy(0, 0).start()
            idx_copy(0, 0).wait()
            gathers(0, 0)
            if gt > 1:
                idx_copy(1, 1).start()
            for t in range(gt):
                p = t & 1
                wait_gathers(t, p)
                if t + 1 < gt:
                    idx_copy(t + 1, p ^ 1).wait()
                    gathers(t + 1, p ^ 1)
                if t + 2 < gt:
                    idx_copy(t + 2, p).start()
                pool(t, p)

            pltpu.sync_copy(out_v, out_hbm.at[pl.ds(b0, BLOCK_BAGS), :])

    return k(table_flat, idx3)


def kernel(indices, tables):
    v = jnp.transpose(indices, (0, 2, 1)).astype(jnp.int32)
    k = (
        (v >= S).astype(jnp.int32)
        + (v >= 2 * S).astype(jnp.int32)
        + (v >= 3 * S).astype(jnp.int32)
    )
    g = jnp.where(
        v < 3 * S,
        (v - k * S) * 4 + k,
        jnp.where(
            v < B3,
            (v - 3 * S) * 4 + 3,
            (v - OV3 + (NSTEP - 2) * LSTEP) * 4
            + jnp.where(v < TAIL0, 3, 0),
        ),
    )
    t_glob = jnp.arange(NUM_TABLES, dtype=jnp.int32)
    t_start = jnp.zeros((NUM_TABLES,), jnp.int32)
    for st, sz in zip(GROUP_STARTS, GROUP_SIZES):
        t_start = jnp.where(t_glob >= st, st, t_start)
    t_local = (t_glob - t_start)[:, None, None]
    g = g + t_local * (LINES_T * 4)

    t2 = jnp.transpose(tables, (0, 2, 1))
    outs = []
    for st, sz in zip(GROUP_STARTS, GROUP_SIZES):
        packed = _tc_pack(t2, sz, st)
        table_flat = packed.reshape(sz * LINES_T * 4, DIM)
        outs.append(_sc_embedding_bag(table_flat, g, sz, st))
    return jnp.concatenate(outs, axis=1)
